# baseline (speedup 1.0000x reference)
import functools

import jax
import jax.numpy as jnp
from jax import lax
from jax.experimental import pallas as pl
from jax.experimental.pallas import tpu as pltpu
from jax.experimental.pallas import tpu_sc as plsc

NB = 320000
NA = 10000
HID = 128
BFD = 16
AFD = 128

NC = 2
NS = 16
NW = NC * NS

CH = 64
NCH_LO = 156
NCH_HI = 158

ROWS_PT = 624


@functools.cache
def _sc_mesh():
    return plsc.VectorSubcoreMesh(
        core_axis_name="c", subcore_axis_name="s",
        num_cores=NC, num_subcores=NS)


def _init_body(fb_ref, w_ref, b_ref, out_ref):
    acc = jnp.dot(fb_ref[...], w_ref[...], preferred_element_type=jnp.float32)
    out_ref[...] = jnp.maximum(acc + b_ref[...], 0.0)


def _tc_init(f_bonds, wiT, bi):
    bt = 8000
    return pl.pallas_call(
        _init_body,
        grid=(NB // bt,),
        in_specs=[
            pl.BlockSpec((bt, BFD), lambda i: (i, 0)),
            pl.BlockSpec((BFD, HID), lambda i: (0, 0)),
            pl.BlockSpec((1, HID), lambda i: (0, 0)),
        ],
        out_specs=pl.BlockSpec((bt, HID), lambda i: (i, 0)),
        out_shape=jax.ShapeDtypeStruct((NB, HID), jnp.float32),
    )(f_bonds, wiT, bi)


_GA_BT = 5000


def _ga_body(h_ref, sp_ref, w_ref, b_ref, g_ref, a_ref):
    i = pl.program_id(0)
    g_ref[...] = jnp.dot(h_ref[...], w_ref[...],
                         preferred_element_type=jnp.float32)

    @pl.when(i < NA // _GA_BT)
    def _():
        s = sp_ref[0] + sp_ref[1]
        a_ref[...] = jnp.dot(s, w_ref[...],
                             preferred_element_type=jnp.float32) + b_ref[...]


def _tc_ga(h_bonds, s_part, whT, bh):
    bt = _GA_BT
    na_blocks = NA // bt
    return pl.pallas_call(
        _ga_body,
        grid=(NB // bt,),
        in_specs=[
            pl.BlockSpec((bt, HID), lambda i: (i, 0)),
            pl.BlockSpec((2, bt, HID),
                         lambda i: (0, jnp.minimum(i, NA // _GA_BT - 1), 0)),
            pl.BlockSpec((HID, HID), lambda i: (0, 0)),
            pl.BlockSpec((1, HID), lambda i: (0, 0)),
        ],
        out_specs=[
            pl.BlockSpec((bt, HID), lambda i: (i, 0)),
            pl.BlockSpec((bt, HID),
                         lambda i: (jnp.minimum(i, NA // _GA_BT - 1), 0)),
        ],
        out_shape=[
            jax.ShapeDtypeStruct((NB, HID), jnp.float32),
            jax.ShapeDtypeStruct((NA, HID), jnp.float32),
        ],
    )(h_bonds, s_part, whT, bh)


def _final_body(fa_ref, sp_ref, w1_ref, w2_ref, b_ref, out_ref):
    s = sp_ref[0] + sp_ref[1]
    acc = jnp.dot(fa_ref[...], w1_ref[...], preferred_element_type=jnp.float32)
    acc += jnp.dot(s, w2_ref[...], preferred_element_type=jnp.float32)
    out_ref[...] = jnp.maximum(acc + b_ref[...], 0.0)


def _tc_final(f_atoms, s_part, wo1T, wo2T, bo):
    return pl.pallas_call(
        _final_body,
        out_shape=jax.ShapeDtypeStruct((NA, HID), jnp.float32),
    )(f_atoms, s_part, wo1T, wo2T, bo)


def _tile_chunks(wid):
    cstart = NCH_LO * wid + 2 * jnp.minimum(wid, 4)
    nchunks = jnp.where(wid < 4, NCH_HI, NCH_LO)
    return cstart, nchunks


def _zero_accum(s_sh, stage, sid):
    def fill_row(r, c):
        for j in range(HID // 16):
            stage[r, pl.ds(j * 16, 16)] = jnp.zeros((16,), jnp.float32)
        return c
    lax.fori_loop(0, CH, fill_row, None)
    for j in range(ROWS_PT // CH):
        pltpu.sync_copy(stage, s_sh.at[pl.ds(sid * ROWS_PT + j * CH, CH)])
    rem = ROWS_PT % CH
    pltpu.sync_copy(stage.at[pl.ds(0, rem)],
                    s_sh.at[pl.ds(sid * ROWS_PT + ROWS_PT - rem, rem)])

    @pl.when(sid == NS - 1)
    def _():
        tail = NA - NS * ROWS_PT
        pltpu.sync_copy(stage.at[pl.ds(0, tail)],
                        s_sh.at[pl.ds(NS * ROWS_PT, tail)])


def _copy_out_accum(s_sh, stage, out_hbm, cid, sid):
    def move(r0, n):
        pltpu.sync_copy(s_sh.at[pl.ds(r0, n)], stage.at[pl.ds(0, n)])
        pltpu.sync_copy(stage.at[pl.ds(0, n)], out_hbm.at[cid].at[pl.ds(r0, n)])

    for j in range(ROWS_PT // CH):
        move(sid * ROWS_PT + j * CH, CH)
    rem = ROWS_PT % CH
    move(sid * ROWS_PT + ROWS_PT - rem, rem)

    @pl.when(sid == NS - 1)
    def _():
        move(NS * ROWS_PT, NA - NS * ROWS_PT)


@functools.cache
def _sc_scatter_kernel():
    bufs = []
    for _ in range(2):
        bufs += [
            pltpu.VMEM((CH,), jnp.int32),
            pltpu.VMEM((CH,), jnp.int32),
            pltpu.VMEM((CH, HID), jnp.float32),
        ]
    return pl.kernel(
        _sc_scatter_body,
        out_type=(
            jax.ShapeDtypeStruct((NC, NA, HID), jnp.float32),
            jax.ShapeDtypeStruct((NB,), jnp.int32),
        ),
        mesh=_sc_mesh(),
        scratch_types=bufs + [
            pltpu.VMEM_SHARED((NA, HID), jnp.float32),
        ] + [pltpu.SemaphoreType.DMA] * 10,
    )


def _sc_scatter_body(h_hbm, b2a_hbm, b2revb_hbm, out_hbm, dout_hbm,
                     idxr0, dest0, rows0, idxr1, dest1, rows1, s_sh,
                     si0, sd0, sr0, ss0, sq0, si1, sd1, sr1, ss1, sq1):
    cid = lax.axis_index("c")
    sid = lax.axis_index("s")
    wid = sid * NC + cid
    cstart, nchunks = _tile_chunks(wid)
    s0 = dict(idxr=idxr0, dest=dest0, rows=rows0,
              si=si0, sd=sd0, sr=sr0, ss=ss0, sq=sq0)
    s1 = dict(idxr=idxr1, dest=dest1, rows=rows1,
              si=si1, sd=sd1, sr=sr1, ss=ss1, sq=sq1)

    def issue_destw(k, s):
        base = (cstart + k) * CH
        pltpu.async_copy(s['dest'], dout_hbm.at[pl.ds(base, CH)], s['sq'])

    def wait_destw(s):
        pltpu.make_async_copy(
            s['dest'], dout_hbm.at[pl.ds(0, CH)], s['sq']).wait()

    def issue_idx(k, s):
        base = (cstart + jnp.minimum(k, nchunks - 1)) * CH
        pltpu.async_copy(b2revb_hbm.at[pl.ds(base, CH)], s['idxr'], s['si'])

    def wait_idx(s):
        pltpu.make_async_copy(
            b2revb_hbm.at[pl.ds(0, CH)], s['idxr'], s['si']).wait()

    def issue_gathers(k, s):
        base = (cstart + k) * CH
        pltpu.async_copy(b2a_hbm.at[s['idxr']], s['dest'], s['sd'])
        pltpu.async_copy(h_hbm.at[pl.ds(base, CH)], s['rows'], s['sr'])

    def wait_gathers(s):
        pltpu.make_async_copy(b2a_hbm.at[s['idxr']], s['dest'], s['sd']).wait()
        pltpu.make_async_copy(h_hbm.at[pl.ds(0, CH)], s['rows'], s['sr']).wait()

    def issue_scatter(s):
        pltpu.async_copy(s['rows'], s_sh.at[s['dest']], s['ss'], add=True)

    def wait_scatter(s):
        pltpu.make_async_copy(s['rows'], s_sh.at[s['dest']], s['ss']).wait()

    _zero_accum(s_sh, rows0, sid)
    plsc.subcore_barrier()

    issue_idx(0, s0)
    issue_idx(1, s1)
    wait_idx(s0)
    issue_gathers(0, s0)
    wait_idx(s1)
    issue_gathers(1, s1)
    wait_gathers(s0)
    issue_destw(0, s0)
    issue_idx(2, s0)
    issue_scatter(s0)

    def phase(k, cur, nxt):
        wait_scatter(nxt)
        wait_destw(nxt)
        wait_idx(nxt)
        issue_gathers(k + 1, nxt)
        wait_gathers(cur)
        issue_destw(k, cur)
        issue_idx(k + 2, cur)
        issue_scatter(cur)

    def pair(i, c):
        phase(2 * i + 1, s1, s0)
        phase(2 * i + 2, s0, s1)
        return c
    lax.fori_loop(0, (nchunks - 2) // 2, pair, None)

    wait_scatter(s0)
    wait_destw(s0)
    wait_gathers(s1)
    issue_destw(nchunks - 1, s1)
    issue_scatter(s1)
    wait_idx(s0)
    wait_scatter(s1)
    wait_destw(s1)

    plsc.subcore_barrier()
    _copy_out_accum(s_sh, rows0, out_hbm, cid, sid)


@functools.cache
def _sc_combine_kernel():
    bufs = []
    for _ in range(2):
        bufs += [
            pltpu.VMEM((CH,), jnp.int32),
            pltpu.VMEM((CH,), jnp.int32),
            pltpu.VMEM((CH,), jnp.int32),
            pltpu.VMEM((CH, HID), jnp.float32),
            pltpu.VMEM((CH, HID), jnp.float32),
            pltpu.VMEM((CH, HID), jnp.float32),
        ]
    return pl.kernel(
        _sc_combine_body,
        out_type=(
            jax.ShapeDtypeStruct((NB, HID), jnp.float32),
            jax.ShapeDtypeStruct((NC, NA, HID), jnp.float32),
        ),
        mesh=_sc_mesh(),
        scratch_types=bufs + [
            pltpu.VMEM_SHARED((NA, HID), jnp.float32),
        ] + [pltpu.SemaphoreType.DMA] * 16,
    )


def _sc_combine_body(h_hbm, g_hbm, a_hbm, b2a_hbm, b2revb_hbm, dest_hbm,
                     hnew_hbm, out_hbm,
                     idxa0, idxr0, dest0, a0, g0, h0,
                     idxa1, idxr1, dest1, a1, g1, h1,
                     s_sh,
                     sia0, sir0, sa0, sg0, sd0, sh0, sw0, ss0,
                     sia1, sir1, sa1, sg1, sd1, sh1, sw1, ss1):
    cid = lax.axis_index("c")
    sid = lax.axis_index("s")
    wid = sid * NC + cid
    cstart, nchunks = _tile_chunks(wid)
    s0 = dict(idxa=idxa0, idxr=idxr0, dest=dest0, a=a0, g=g0, h=h0,
              sia=sia0, sir=sir0, sa=sa0, sg=sg0, sd=sd0, sh=sh0,
              sw=sw0, ss=ss0)
    s1 = dict(idxa=idxa1, idxr=idxr1, dest=dest1, a=a1, g=g1, h=h1,
              sia=sia1, sir=sir1, sa=sa1, sg=sg1, sd=sd1, sh=sh1,
              sw=sw1, ss=ss1)

    def issue_idx(k, s):
        base = (cstart + jnp.minimum(k, nchunks - 1)) * CH
        pltpu.async_copy(b2a_hbm.at[pl.ds(base, CH)], s['idxa'], s['sia'])
        pltpu.async_copy(b2revb_hbm.at[pl.ds(base, CH)], s['idxr'], s['sir'])

    def wait_idx(s):
        pltpu.make_async_copy(
            b2a_hbm.at[pl.ds(0, CH)], s['idxa'], s['sia']).wait()
        pltpu.make_async_copy(
            b2revb_hbm.at[pl.ds(0, CH)], s['idxr'], s['sir']).wait()

    def issue_gathers(k, s):
        base = (cstart + k) * CH
        pltpu.async_copy(a_hbm.at[s['idxa']], s['a'], s['sa'])
        pltpu.async_copy(g_hbm.at[s['idxr']], s['g'], s['sg'])
        pltpu.async_copy(dest_hbm.at[pl.ds(base, CH)], s['dest'], s['sd'])
        pltpu.async_copy(h_hbm.at[pl.ds(base, CH)], s['h'], s['sh'])

    def wait_gathers(s):
        pltpu.make_async_copy(a_hbm.at[s['idxa']], s['a'], s['sa']).wait()
        pltpu.make_async_copy(g_hbm.at[s['idxr']], s['g'], s['sg']).wait()
        pltpu.make_async_copy(
            dest_hbm.at[pl.ds(0, CH)], s['dest'], s['sd']).wait()
        pltpu.make_async_copy(h_hbm.at[pl.ds(0, CH)], s['h'], s['sh']).wait()

    def compute(s):
        h, a, g = s['h'], s['a'], s['g']

        def row_pair(q, c):
            r2 = pl.multiple_of(q * 2, 2)
            rows = pl.ds(r2, 2)
            for j in range(HID // 16):
                sl = pl.ds(j * 16, 16)
                h[rows, sl] = jnp.maximum(
                    h[rows, sl] + a[rows, sl] - g[rows, sl], 0.0)
            return c
        lax.fori_loop(0, CH // 2, row_pair, None)

    def issue_writes(k, s):
        base = (cstart + k) * CH
        pltpu.async_copy(s['h'], hnew_hbm.at[pl.ds(base, CH)], s['sw'])
        pltpu.async_copy(s['h'], s_sh.at[s['dest']], s['ss'], add=True)

    def wait_writes(s):
        pltpu.make_async_copy(s['h'], hnew_hbm.at[pl.ds(0, CH)], s['sw']).wait()
        pltpu.make_async_copy(s['h'], s_sh.at[s['dest']], s['ss']).wait()

    _zero_accum(s_sh, h0, sid)
    plsc.subcore_barrier()

    issue_idx(0, s0)
    issue_idx(1, s1)
    wait_idx(s0)
    issue_gathers(0, s0)
    wait_idx(s1)
    issue_gathers(1, s1)
    wait_gathers(s0)
    issue_idx(2, s0)
    compute(s0)
    issue_writes(0, s0)

    def phase(k, cur, nxt):
        wait_writes(nxt)
        wait_idx(nxt)
        issue_gathers(k + 1, nxt)
        wait_gathers(cur)
        issue_idx(k + 2, cur)
        compute(cur)
        issue_writes(k, cur)

    def pair(i, c):
        phase(2 * i + 1, s1, s0)
        phase(2 * i + 2, s0, s1)
        return c
    lax.fori_loop(0, (nchunks - 2) // 2, pair, None)

    wait_writes(s0)
    wait_gathers(s1)
    compute(s1)
    issue_writes(nchunks - 1, s1)
    wait_idx(s0)
    wait_writes(s1)

    plsc.subcore_barrier()
    _copy_out_accum(s_sh, h0, out_hbm, cid, sid)


def kernel(f_atoms, f_bonds, b2a, b2revb,
           Wi_w, Wi_b, Wh_w, Wh_b, Wo_w, Wo_b):
    wiT = Wi_w.T
    whT = Wh_w.T
    wo1T = Wo_w[:, :AFD].T
    wo2T = Wo_w[:, AFD:].T
    bi = Wi_b.reshape(1, HID)
    bh = Wh_b.reshape(1, HID)
    bo = Wo_b.reshape(1, HID)

    h_bonds = _tc_init(f_bonds, wiT, bi)
    s_part, dest = _sc_scatter_kernel()(h_bonds, b2a, b2revb)
    for _ in range(2):
        g, a = _tc_ga(h_bonds, s_part, whT, bh)
        h_bonds, s_part = _sc_combine_kernel()(
            h_bonds, g, a, b2a, b2revb, dest)
    h_atoms = _tc_final(f_atoms, s_part, wo1T, wo2T, bo)
    return (h_atoms, h_bonds)


# speedup vs baseline: 1.0516x; 1.0516x over previous
import functools

import jax
import jax.numpy as jnp
from jax import lax
from jax.experimental import pallas as pl
from jax.experimental.pallas import tpu as pltpu
from jax.experimental.pallas import tpu_sc as plsc

NB = 320000
NA = 10000
HID = 128
BFD = 16
AFD = 128

NC = 2
NS = 16
NW = NC * NS

CH = 64
NCH_LO = 156
NCH_HI = 158

ROWS_PT = 624


@functools.cache
def _sc_mesh():
    return plsc.VectorSubcoreMesh(
        core_axis_name="c", subcore_axis_name="s",
        num_cores=NC, num_subcores=NS)


def _init_body(fb_ref, w_ref, b_ref, out_ref):
    acc = jnp.dot(fb_ref[...], w_ref[...], preferred_element_type=jnp.float32)
    out_ref[...] = jnp.maximum(acc + b_ref[...], 0.0)


def _tc_init(f_bonds, wiT, bi):
    bt = 8000
    return pl.pallas_call(
        _init_body,
        grid=(NB // bt,),
        in_specs=[
            pl.BlockSpec((bt, BFD), lambda i: (i, 0)),
            pl.BlockSpec((BFD, HID), lambda i: (0, 0)),
            pl.BlockSpec((1, HID), lambda i: (0, 0)),
        ],
        out_specs=pl.BlockSpec((bt, HID), lambda i: (i, 0)),
        out_shape=jax.ShapeDtypeStruct((NB, HID), jnp.float32),
    )(f_bonds, wiT, bi)


def _g_body(h_ref, w_ref, out_ref):
    out_ref[...] = jnp.dot(h_ref[...], w_ref[...],
                           preferred_element_type=jnp.float32)


def _tc_g(h_bonds, whT):
    bt = 8000
    return pl.pallas_call(
        _g_body,
        grid=(NB // bt,),
        in_specs=[
            pl.BlockSpec((bt, HID), lambda i: (i, 0)),
            pl.BlockSpec((HID, HID), lambda i: (0, 0)),
        ],
        out_specs=pl.BlockSpec((bt, HID), lambda i: (i, 0)),
        out_shape=jax.ShapeDtypeStruct((NB, HID), jnp.float32),
    )(h_bonds, whT)


def _a_body(sp_ref, w_ref, b_ref, out_ref):
    s = sp_ref[0] + sp_ref[1]
    out_ref[...] = jnp.dot(s, w_ref[...],
                           preferred_element_type=jnp.float32) + b_ref[...]


def _tc_a(s_part, whT, bh):
    return pl.pallas_call(
        _a_body,
        out_shape=jax.ShapeDtypeStruct((NA, HID), jnp.float32),
    )(s_part, whT, bh)


def _final_body(fa_ref, sp_ref, w1_ref, w2_ref, b_ref, out_ref):
    s = sp_ref[0] + sp_ref[1]
    acc = jnp.dot(fa_ref[...], w1_ref[...], preferred_element_type=jnp.float32)
    acc += jnp.dot(s, w2_ref[...], preferred_element_type=jnp.float32)
    out_ref[...] = jnp.maximum(acc + b_ref[...], 0.0)


def _tc_final(f_atoms, s_part, wo1T, wo2T, bo):
    return pl.pallas_call(
        _final_body,
        out_shape=jax.ShapeDtypeStruct((NA, HID), jnp.float32),
    )(f_atoms, s_part, wo1T, wo2T, bo)


def _tile_chunks(wid):
    cstart = NCH_LO * wid + 2 * jnp.minimum(wid, 4)
    nchunks = jnp.where(wid < 4, NCH_HI, NCH_LO)
    return cstart, nchunks


def _zero_accum(s_sh, stage, sid):
    def fill_row(r, c):
        for j in range(HID // 16):
            stage[r, pl.ds(j * 16, 16)] = jnp.zeros((16,), jnp.float32)
        return c
    lax.fori_loop(0, CH, fill_row, None)
    for j in range(ROWS_PT // CH):
        pltpu.sync_copy(stage, s_sh.at[pl.ds(sid * ROWS_PT + j * CH, CH)])
    rem = ROWS_PT % CH
    pltpu.sync_copy(stage.at[pl.ds(0, rem)],
                    s_sh.at[pl.ds(sid * ROWS_PT + ROWS_PT - rem, rem)])

    @pl.when(sid == NS - 1)
    def _():
        tail = NA - NS * ROWS_PT
        pltpu.sync_copy(stage.at[pl.ds(0, tail)],
                        s_sh.at[pl.ds(NS * ROWS_PT, tail)])


def _copy_out_accum(s_sh, stage, out_hbm, cid, sid):
    def move(r0, n):
        pltpu.sync_copy(s_sh.at[pl.ds(r0, n)], stage.at[pl.ds(0, n)])
        pltpu.sync_copy(stage.at[pl.ds(0, n)], out_hbm.at[cid].at[pl.ds(r0, n)])

    for j in range(ROWS_PT // CH):
        move(sid * ROWS_PT + j * CH, CH)
    rem = ROWS_PT % CH
    move(sid * ROWS_PT + ROWS_PT - rem, rem)

    @pl.when(sid == NS - 1)
    def _():
        move(NS * ROWS_PT, NA - NS * ROWS_PT)


@functools.cache
def _sc_scatter_kernel():
    bufs = []
    for _ in range(2):
        bufs += [
            pltpu.VMEM((CH,), jnp.int32),
            pltpu.VMEM((CH,), jnp.int32),
            pltpu.VMEM((CH, HID), jnp.float32),
        ]
    return pl.kernel(
        _sc_scatter_body,
        out_type=(
            jax.ShapeDtypeStruct((NC, NA, HID), jnp.float32),
            jax.ShapeDtypeStruct((NB,), jnp.int32),
        ),
        mesh=_sc_mesh(),
        scratch_types=bufs + [
            pltpu.VMEM_SHARED((NA, HID), jnp.float32),
        ] + [pltpu.SemaphoreType.DMA] * 10,
    )


def _sc_scatter_body(h_hbm, b2a_hbm, b2revb_hbm, out_hbm, dout_hbm,
                     idxr0, dest0, rows0, idxr1, dest1, rows1, s_sh,
                     si0, sd0, sr0, ss0, sq0, si1, sd1, sr1, ss1, sq1):
    cid = lax.axis_index("c")
    sid = lax.axis_index("s")
    wid = sid * NC + cid
    cstart, nchunks = _tile_chunks(wid)
    s0 = dict(idxr=idxr0, dest=dest0, rows=rows0,
              si=si0, sd=sd0, sr=sr0, ss=ss0, sq=sq0)
    s1 = dict(idxr=idxr1, dest=dest1, rows=rows1,
              si=si1, sd=sd1, sr=sr1, ss=ss1, sq=sq1)

    def issue_destw(k, s):
        base = (cstart + k) * CH
        pltpu.async_copy(s['dest'], dout_hbm.at[pl.ds(base, CH)], s['sq'])

    def wait_destw(s):
        pltpu.make_async_copy(
            s['dest'], dout_hbm.at[pl.ds(0, CH)], s['sq']).wait()

    def issue_idx(k, s):
        base = (cstart + jnp.minimum(k, nchunks - 1)) * CH
        pltpu.async_copy(b2revb_hbm.at[pl.ds(base, CH)], s['idxr'], s['si'])

    def wait_idx(s):
        pltpu.make_async_copy(
            b2revb_hbm.at[pl.ds(0, CH)], s['idxr'], s['si']).wait()

    def issue_gathers(k, s):
        base = (cstart + k) * CH
        pltpu.async_copy(b2a_hbm.at[s['idxr']], s['dest'], s['sd'])
        pltpu.async_copy(h_hbm.at[pl.ds(base, CH)], s['rows'], s['sr'])

    def wait_gathers(s):
        pltpu.make_async_copy(b2a_hbm.at[s['idxr']], s['dest'], s['sd']).wait()
        pltpu.make_async_copy(h_hbm.at[pl.ds(0, CH)], s['rows'], s['sr']).wait()

    def issue_scatter(s):
        pltpu.async_copy(s['rows'], s_sh.at[s['dest']], s['ss'], add=True)

    def wait_scatter(s):
        pltpu.make_async_copy(s['rows'], s_sh.at[s['dest']], s['ss']).wait()

    _zero_accum(s_sh, rows0, sid)
    plsc.subcore_barrier()

    issue_idx(0, s0)
    issue_idx(1, s1)
    wait_idx(s0)
    issue_gathers(0, s0)
    wait_idx(s1)
    issue_gathers(1, s1)
    wait_gathers(s0)
    issue_destw(0, s0)
    issue_idx(2, s0)
    issue_scatter(s0)

    def phase(k, cur, nxt):
        wait_scatter(nxt)
        wait_destw(nxt)
        wait_idx(nxt)
        issue_gathers(k + 1, nxt)
        wait_gathers(cur)
        issue_destw(k, cur)
        issue_idx(k + 2, cur)
        issue_scatter(cur)

    def pair(i, c):
        phase(2 * i + 1, s1, s0)
        phase(2 * i + 2, s0, s1)
        return c
    lax.fori_loop(0, (nchunks - 2) // 2, pair, None)

    wait_scatter(s0)
    wait_destw(s0)
    wait_gathers(s1)
    issue_destw(nchunks - 1, s1)
    issue_scatter(s1)
    wait_idx(s0)
    wait_scatter(s1)
    wait_destw(s1)

    plsc.subcore_barrier()
    _copy_out_accum(s_sh, rows0, out_hbm, cid, sid)


@functools.cache
def _sc_combine_kernel():
    bufs = []
    for _ in range(2):
        bufs += [
            pltpu.VMEM((CH,), jnp.int32),
            pltpu.VMEM((CH,), jnp.int32),
            pltpu.VMEM((CH,), jnp.int32),
            pltpu.VMEM((CH, HID), jnp.float32),
            pltpu.VMEM((CH, HID), jnp.float32),
            pltpu.VMEM((CH, HID), jnp.float32),
        ]
    return pl.kernel(
        _sc_combine_body,
        out_type=(
            jax.ShapeDtypeStruct((NB, HID), jnp.float32),
            jax.ShapeDtypeStruct((NC, NA, HID), jnp.float32),
        ),
        mesh=_sc_mesh(),
        scratch_types=bufs + [
            pltpu.VMEM_SHARED((NA, HID), jnp.float32),
        ] + [pltpu.SemaphoreType.DMA] * 16,
    )


def _sc_combine_body(h_hbm, g_hbm, a_hbm, b2a_hbm, b2revb_hbm, dest_hbm,
                     hnew_hbm, out_hbm,
                     idxa0, idxr0, dest0, a0, g0, h0,
                     idxa1, idxr1, dest1, a1, g1, h1,
                     s_sh,
                     sia0, sir0, sa0, sg0, sd0, sh0, sw0, ss0,
                     sia1, sir1, sa1, sg1, sd1, sh1, sw1, ss1):
    cid = lax.axis_index("c")
    sid = lax.axis_index("s")
    wid = sid * NC + cid
    cstart, nchunks = _tile_chunks(wid)
    s0 = dict(idxa=idxa0, idxr=idxr0, dest=dest0, a=a0, g=g0, h=h0,
              sia=sia0, sir=sir0, sa=sa0, sg=sg0, sd=sd0, sh=sh0,
              sw=sw0, ss=ss0)
    s1 = dict(idxa=idxa1, idxr=idxr1, dest=dest1, a=a1, g=g1, h=h1,
              sia=sia1, sir=sir1, sa=sa1, sg=sg1, sd=sd1, sh=sh1,
              sw=sw1, ss=ss1)

    def issue_idx(k, s):
        base = (cstart + jnp.minimum(k, nchunks - 1)) * CH
        pltpu.async_copy(b2a_hbm.at[pl.ds(base, CH)], s['idxa'], s['sia'])
        pltpu.async_copy(b2revb_hbm.at[pl.ds(base, CH)], s['idxr'], s['sir'])

    def wait_idx(s):
        pltpu.make_async_copy(
            b2a_hbm.at[pl.ds(0, CH)], s['idxa'], s['sia']).wait()
        pltpu.make_async_copy(
            b2revb_hbm.at[pl.ds(0, CH)], s['idxr'], s['sir']).wait()

    def issue_gathers(k, s):
        base = (cstart + k) * CH
        pltpu.async_copy(a_hbm.at[s['idxa']], s['a'], s['sa'])
        pltpu.async_copy(g_hbm.at[s['idxr']], s['g'], s['sg'])
        pltpu.async_copy(dest_hbm.at[pl.ds(base, CH)], s['dest'], s['sd'])
        pltpu.async_copy(h_hbm.at[pl.ds(base, CH)], s['h'], s['sh'])

    def wait_gathers(s):
        pltpu.make_async_copy(a_hbm.at[s['idxa']], s['a'], s['sa']).wait()
        pltpu.make_async_copy(g_hbm.at[s['idxr']], s['g'], s['sg']).wait()
        pltpu.make_async_copy(
            dest_hbm.at[pl.ds(0, CH)], s['dest'], s['sd']).wait()
        pltpu.make_async_copy(h_hbm.at[pl.ds(0, CH)], s['h'], s['sh']).wait()

    def compute(s):
        h, a, g = s['h'], s['a'], s['g']

        def row_pair(q, c):
            r2 = pl.multiple_of(q * 2, 2)
            rows = pl.ds(r2, 2)
            for j in range(HID // 16):
                sl = pl.ds(j * 16, 16)
                h[rows, sl] = jnp.maximum(
                    h[rows, sl] + a[rows, sl] - g[rows, sl], 0.0)
            return c
        lax.fori_loop(0, CH // 2, row_pair, None)

    def issue_writes(k, s):
        base = (cstart + k) * CH
        pltpu.async_copy(s['h'], hnew_hbm.at[pl.ds(base, CH)], s['sw'])
        pltpu.async_copy(s['h'], s_sh.at[s['dest']], s['ss'], add=True)

    def wait_writes(s):
        pltpu.make_async_copy(s['h'], hnew_hbm.at[pl.ds(0, CH)], s['sw']).wait()
        pltpu.make_async_copy(s['h'], s_sh.at[s['dest']], s['ss']).wait()

    _zero_accum(s_sh, h0, sid)
    plsc.subcore_barrier()

    issue_idx(0, s0)
    issue_idx(1, s1)
    wait_idx(s0)
    issue_gathers(0, s0)
    wait_idx(s1)
    issue_gathers(1, s1)
    wait_gathers(s0)
    issue_idx(2, s0)
    compute(s0)
    issue_writes(0, s0)

    def phase(k, cur, nxt):
        wait_writes(nxt)
        wait_idx(nxt)
        issue_gathers(k + 1, nxt)
        wait_gathers(cur)
        issue_idx(k + 2, cur)
        compute(cur)
        issue_writes(k, cur)

    def pair(i, c):
        phase(2 * i + 1, s1, s0)
        phase(2 * i + 2, s0, s1)
        return c
    lax.fori_loop(0, (nchunks - 2) // 2, pair, None)

    wait_writes(s0)
    wait_gathers(s1)
    compute(s1)
    issue_writes(nchunks - 1, s1)
    wait_idx(s0)
    wait_writes(s1)

    plsc.subcore_barrier()
    _copy_out_accum(s_sh, h0, out_hbm, cid, sid)


def kernel(f_atoms, f_bonds, b2a, b2revb,
           Wi_w, Wi_b, Wh_w, Wh_b, Wo_w, Wo_b):
    wiT = Wi_w.T
    whT = Wh_w.T
    wo1T = Wo_w[:, :AFD].T
    wo2T = Wo_w[:, AFD:].T
    bi = Wi_b.reshape(1, HID)
    bh = Wh_b.reshape(1, HID)
    bo = Wo_b.reshape(1, HID)

    h_bonds = _tc_init(f_bonds, wiT, bi)
    s_part, dest = _sc_scatter_kernel()(h_bonds, b2a, b2revb)
    for _ in range(2):
        a = _tc_a(s_part, whT, bh)
        g = _tc_g(h_bonds, whT)
        h_bonds, s_part = _sc_combine_kernel()(
            h_bonds, g, a, b2a, b2revb, dest)
    h_atoms = _tc_final(f_atoms, s_part, wo1T, wo2T, bo)
    return (h_atoms, h_bonds)
